# baseline (speedup 1.0000x reference)
import functools

import jax
import jax.numpy as jnp
from jax import lax
from jax.experimental import pallas as pl
from jax.experimental.pallas import tpu as pltpu
from jax.experimental.pallas import tpu_sc as plsc

N = 4096
K = 32
RB = 512
NB = N // RB
DPAD = 16


def _topk_block(s_ref, sT_ref, idx_ref, i):
    px = s_ref[pl.ds(i * RB, RB), 0:1]
    py = s_ref[pl.ds(i * RB, RB), 1:2]
    qx = sT_ref[0:1, :]
    qy = sT_ref[1:2, :]
    dx = px - qx
    dy = py - qy
    d = jnp.sqrt((dx * dx + 1e-6) + (dy * dy + 1e-6))
    lane = lax.broadcasted_iota(jnp.int32, (RB, N), 1)
    big = jnp.int32(2 ** 30)
    inf = jnp.float32(float("inf"))
    del big
    cols = []
    for _ in range(K):
        sel = jnp.argmin(d, axis=1).astype(jnp.int32)[:, None]
        cols.append(sel)
        d = jnp.where(lane == sel, inf, d)
    idx_ref[...] = jnp.concatenate(cols, axis=1)


def _topk(s, sT, half):
    def body(s_ref, sT_ref, idx_ref):
        _topk_block(s_ref, sT_ref, idx_ref,
                    pl.program_id(0) + half * (NB // 2))
    return pl.pallas_call(
        body,
        grid=(NB // 2,),
        in_specs=[
            pl.BlockSpec((N, 4), lambda i: (0, 0)),
            pl.BlockSpec((4, N), lambda i: (0, 0)),
        ],
        out_specs=pl.BlockSpec((RB, K), lambda i: (i, 0)),
        out_shape=jax.ShapeDtypeStruct((N // 2, K), jnp.int32),
    )(s, sT)


def _sc_gather(idx_flat, sT_flat, half):
    info = plsc.get_sparse_core_info()
    nc, ns = info.num_cores, info.num_subcores
    nw = nc * ns
    b = idx_flat.shape[0]
    bpw = b // nw
    nchunk = bpw // 16
    mesh = plsc.VectorSubcoreMesh(core_axis_name="c", subcore_axis_name="s")

    @functools.partial(
        pl.kernel, mesh=mesh,
        out_type=jax.ShapeDtypeStruct((b * DPAD,), jnp.float32),
        scratch_types=[
            pltpu.VMEM((4 * N,), jnp.float32),
            pltpu.VMEM((bpw,), jnp.int32),
            pltpu.VMEM((bpw * DPAD,), jnp.float32),
        ],
        compiler_params=pltpu.CompilerParams(needs_layout_passes=False),
    )
    def k(sT_hbm, idx_hbm, out_hbm, tab_v, idx_v, out_v):
        wid = lax.axis_index("s") * nc + lax.axis_index("c")
        base = wid * bpw
        pltpu.sync_copy(sT_hbm, tab_v)
        pltpu.sync_copy(idx_hbm.at[pl.ds(base, bpw)], idx_v)
        lanes = lax.broadcasted_iota(jnp.int32, (16,), 0)
        zeros = jnp.zeros((16,), jnp.float32)
        ones = zeros + 1.0

        def chunk(q, carry):
            cb = q * 16
            row = (half * b + base + cb) >> 5
            rowv = lanes * 0 + row
            idxv = idx_v[pl.ds(cb, 16)]
            pvec = (cb + lanes) * DPAD
            rel = []
            for c in range(4):
                sj = plsc.load_gather(tab_v, [idxv + c * N])
                si = plsc.load_gather(tab_v, [rowv + c * N])
                r = si - sj
                rel.append(r)
                plsc.store_scatter(out_v, [pvec + c], r)
            eye = jnp.where(idxv == rowv, ones, zeros)
            plsc.store_scatter(out_v, [pvec + 4], eye)
            msq = rel[0] * rel[0] + rel[1] * rel[1]
            mrad = jnp.where(msq < 1.0, ones, zeros)
            plsc.store_scatter(out_v, [pvec + 5], mrad)
            return carry

        lax.fori_loop(0, nchunk, chunk, 0)
        pltpu.sync_copy(out_v, out_hbm.at[pl.ds(base * DPAD, bpw * DPAD)])

    return k(sT_flat, idx_flat)


def _mlp_body(half, gath_ref, idx_ref, s_ref, g_ref,
              w1t_ref, b1_ref, w2t_ref, b2_ref,
              wf1t_ref, bf1_ref, wf2t_ref, bf2_ref,
              wf3t_ref, bf3_ref, wf4t_ref, bf4_ref, out_ref):
    del half, idx_ref
    s_blk = s_ref[...]
    g_blk = g_ref[...]
    gath = gath_ref[...].reshape(RB, K, DPAD)
    mask = gath[:, :, 5]
    x2 = gath[:, :, 0:5].reshape(RB * K, 5)
    h1 = jax.nn.relu(jnp.dot(x2, w1t_ref[...],
                             preferred_element_type=jnp.float32) + b1_ref[...])
    h2 = jax.nn.relu(jnp.dot(h1, w2t_ref[...],
                             preferred_element_type=jnp.float32) + b2_ref[...])
    hm = jnp.max(h2.reshape(RB, K, 128) * mask[:, :, None], axis=1)

    sg = s_blk[:, 0:2] - g_blk
    sv = s_blk[:, 2:4]
    feat = jnp.concatenate([hm, sg, sv], axis=1)
    f = jax.nn.relu(jnp.dot(feat, wf1t_ref[...],
                            preferred_element_type=jnp.float32) + bf1_ref[...])
    f = jax.nn.relu(jnp.dot(f, wf2t_ref[...],
                            preferred_element_type=jnp.float32) + bf2_ref[...])
    f = jax.nn.relu(jnp.dot(f, wf3t_ref[...],
                            preferred_element_type=jnp.float32) + bf3_ref[...])
    f = jnp.dot(f, wf4t_ref[...],
                preferred_element_type=jnp.float32) + bf4_ref[...]
    kk = 2.0 * jax.nn.sigmoid(f) + 0.2
    ax = -(kk[:, 0:1] * sg[:, 0:1] + kk[:, 1:2] * sv[:, 0:1])
    ay = -(kk[:, 2:3] * sg[:, 1:2] + kk[:, 3:4] * sv[:, 1:2])
    out_ref[...] = jnp.concatenate([ax, ay], axis=1)


def _mlp(half, gath, idx, s, g, *wb):
    full = lambda shape: pl.BlockSpec(shape, lambda i: tuple(0 for _ in shape))
    wb_specs = [full(w.shape) for w in wb]
    hb = NB // 2
    return pl.pallas_call(
        functools.partial(_mlp_body, half),
        grid=(hb,),
        in_specs=[
            pl.BlockSpec((RB * K, DPAD), lambda i: (i, 0)),
            pl.BlockSpec((RB, K), lambda i: (i, 0)),
            pl.BlockSpec((RB, 4), lambda i, h=half: (i + h * hb, 0)),
            pl.BlockSpec((RB, 2), lambda i, h=half: (i + h * hb, 0)),
        ] + wb_specs,
        out_specs=pl.BlockSpec((RB, 2), lambda i: (i, 0)),
        out_shape=jax.ShapeDtypeStruct((N // 2, 2), jnp.float32),
    )(gath, idx, s, g, *wb)


def kernel(s, g, W1, b1, W2, b2, Wf1, bf1, Wf2, bf2, Wf3, bf3, Wf4, bf4):
    sT = s.T
    sT_flat = sT.reshape(-1)
    wb = (W1.T, b1[None, :], W2.T, b2[None, :],
          Wf1.T, bf1[None, :], Wf2.T, bf2[None, :],
          Wf3.T, bf3[None, :], Wf4.T, bf4[None, :])
    idx0 = _topk(s, sT, 0)
    gath0 = _sc_gather(idx0.reshape(-1), sT_flat, 0).reshape(N * K // 2, DPAD)
    idx1 = _topk(s, sT, 1)
    gath1 = _sc_gather(idx1.reshape(-1), sT_flat, 1).reshape(N * K // 2, DPAD)
    a0 = _mlp(0, gath0, idx0, s, g, *wb)
    a1 = _mlp(1, gath1, idx1, s, g, *wb)
    return jnp.concatenate([a0, a1], axis=0)


# speedup vs baseline: 1.1323x; 1.1323x over previous
import functools

import jax
import jax.numpy as jnp
from jax import lax
from jax.experimental import pallas as pl
from jax.experimental.pallas import tpu as pltpu
from jax.experimental.pallas import tpu_sc as plsc

N = 4096
K = 32
RB = 128
NB = N // RB
DPAD = 16


def _topk_block(s_ref, sT_ref, idx_ref, i):
    px = s_ref[pl.ds(i * RB, RB), 0:1]
    py = s_ref[pl.ds(i * RB, RB), 1:2]
    qx = sT_ref[0:1, :]
    qy = sT_ref[1:2, :]
    dx = px - qx
    dy = py - qy
    d = jnp.sqrt((dx * dx + 1e-6) + (dy * dy + 1e-6))
    lane = lax.broadcasted_iota(jnp.int32, (RB, N), 1)
    big = jnp.int32(2 ** 30)
    inf = jnp.float32(float("inf"))
    del big
    cols = []
    for _ in range(K):
        sel = jnp.argmin(d, axis=1).astype(jnp.int32)[:, None]
        cols.append(sel)
        d = jnp.where(lane == sel, inf, d)
    idx_ref[...] = jnp.concatenate(cols, axis=1)


def _topk(s, sT, half):
    def body(s_ref, sT_ref, idx_ref):
        _topk_block(s_ref, sT_ref, idx_ref,
                    pl.program_id(0) + half * (NB // 2))
    return pl.pallas_call(
        body,
        grid=(NB // 2,),
        in_specs=[
            pl.BlockSpec((N, 4), lambda i: (0, 0)),
            pl.BlockSpec((4, N), lambda i: (0, 0)),
        ],
        out_specs=pl.BlockSpec((RB, K), lambda i: (i, 0)),
        out_shape=jax.ShapeDtypeStruct((N // 2, K), jnp.int32),
    )(s, sT)


def _sc_gather(idx_flat, sT_flat, half):
    info = plsc.get_sparse_core_info()
    nc, ns = info.num_cores, info.num_subcores
    nw = nc * ns
    b = idx_flat.shape[0]
    bpw = b // nw
    nchunk = bpw // 16
    mesh = plsc.VectorSubcoreMesh(core_axis_name="c", subcore_axis_name="s")

    @functools.partial(
        pl.kernel, mesh=mesh,
        out_type=jax.ShapeDtypeStruct((b * DPAD,), jnp.float32),
        scratch_types=[
            pltpu.VMEM((4 * N,), jnp.float32),
            pltpu.VMEM((bpw,), jnp.int32),
            pltpu.VMEM((bpw * DPAD,), jnp.float32),
        ],
        compiler_params=pltpu.CompilerParams(needs_layout_passes=False),
    )
    def k(sT_hbm, idx_hbm, out_hbm, tab_v, idx_v, out_v):
        wid = lax.axis_index("s") * nc + lax.axis_index("c")
        base = wid * bpw
        pltpu.sync_copy(sT_hbm, tab_v)
        pltpu.sync_copy(idx_hbm.at[pl.ds(base, bpw)], idx_v)
        lanes = lax.broadcasted_iota(jnp.int32, (16,), 0)
        zeros = jnp.zeros((16,), jnp.float32)
        ones = zeros + 1.0

        def chunk(q, carry):
            cb = q * 16
            row = (half * b + base + cb) >> 5
            rowv = lanes * 0 + row
            idxv = idx_v[pl.ds(cb, 16)]
            pvec = (cb + lanes) * DPAD
            rel = []
            for c in range(4):
                sj = plsc.load_gather(tab_v, [idxv + c * N])
                si = plsc.load_gather(tab_v, [rowv + c * N])
                r = si - sj
                rel.append(r)
                plsc.store_scatter(out_v, [pvec + c], r)
            eye = jnp.where(idxv == rowv, ones, zeros)
            plsc.store_scatter(out_v, [pvec + 4], eye)
            msq = rel[0] * rel[0] + rel[1] * rel[1]
            mrad = jnp.where(msq < 1.0, ones, zeros)
            plsc.store_scatter(out_v, [pvec + 5], mrad)
            return carry

        lax.fori_loop(0, nchunk, chunk, 0)
        pltpu.sync_copy(out_v, out_hbm.at[pl.ds(base * DPAD, bpw * DPAD)])

    return k(sT_flat, idx_flat)


def _mlp_body(half, gath_ref, idx_ref, s_ref, g_ref,
              w1t_ref, b1_ref, w2t_ref, b2_ref,
              wf1t_ref, bf1_ref, wf2t_ref, bf2_ref,
              wf3t_ref, bf3_ref, wf4t_ref, bf4_ref, out_ref):
    del half, idx_ref
    s_blk = s_ref[...]
    g_blk = g_ref[...]
    gath = gath_ref[...].reshape(RB, K, DPAD)
    mask = gath[:, :, 5]
    x2 = gath[:, :, 0:5].reshape(RB * K, 5)
    h1 = jax.nn.relu(jnp.dot(x2, w1t_ref[...],
                             preferred_element_type=jnp.float32) + b1_ref[...])
    h2 = jax.nn.relu(jnp.dot(h1, w2t_ref[...],
                             preferred_element_type=jnp.float32) + b2_ref[...])
    hm = jnp.max(h2.reshape(RB, K, 128) * mask[:, :, None], axis=1)

    sg = s_blk[:, 0:2] - g_blk
    sv = s_blk[:, 2:4]
    feat = jnp.concatenate([hm, sg, sv], axis=1)
    f = jax.nn.relu(jnp.dot(feat, wf1t_ref[...],
                            preferred_element_type=jnp.float32) + bf1_ref[...])
    f = jax.nn.relu(jnp.dot(f, wf2t_ref[...],
                            preferred_element_type=jnp.float32) + bf2_ref[...])
    f = jax.nn.relu(jnp.dot(f, wf3t_ref[...],
                            preferred_element_type=jnp.float32) + bf3_ref[...])
    f = jnp.dot(f, wf4t_ref[...],
                preferred_element_type=jnp.float32) + bf4_ref[...]
    kk = 2.0 * jax.nn.sigmoid(f) + 0.2
    ax = -(kk[:, 0:1] * sg[:, 0:1] + kk[:, 1:2] * sv[:, 0:1])
    ay = -(kk[:, 2:3] * sg[:, 1:2] + kk[:, 3:4] * sv[:, 1:2])
    out_ref[...] = jnp.concatenate([ax, ay], axis=1)


def _mlp(half, gath, idx, s, g, *wb):
    full = lambda shape: pl.BlockSpec(shape, lambda i: tuple(0 for _ in shape))
    wb_specs = [full(w.shape) for w in wb]
    hb = NB // 2
    return pl.pallas_call(
        functools.partial(_mlp_body, half),
        grid=(hb,),
        in_specs=[
            pl.BlockSpec((RB * K, DPAD), lambda i: (i, 0)),
            pl.BlockSpec((RB, K), lambda i: (i, 0)),
            pl.BlockSpec((RB, 4), lambda i, h=half: (i + h * hb, 0)),
            pl.BlockSpec((RB, 2), lambda i, h=half: (i + h * hb, 0)),
        ] + wb_specs,
        out_specs=pl.BlockSpec((RB, 2), lambda i: (i, 0)),
        out_shape=jax.ShapeDtypeStruct((N // 2, 2), jnp.float32),
    )(gath, idx, s, g, *wb)


def kernel(s, g, W1, b1, W2, b2, Wf1, bf1, Wf2, bf2, Wf3, bf3, Wf4, bf4):
    sT = s.T
    sT_flat = sT.reshape(-1)
    wb = (W1.T, b1[None, :], W2.T, b2[None, :],
          Wf1.T, bf1[None, :], Wf2.T, bf2[None, :],
          Wf3.T, bf3[None, :], Wf4.T, bf4[None, :])
    idx0 = _topk(s, sT, 0)
    gath0 = _sc_gather(idx0.reshape(-1), sT_flat, 0).reshape(N * K // 2, DPAD)
    idx1 = _topk(s, sT, 1)
    gath1 = _sc_gather(idx1.reshape(-1), sT_flat, 1).reshape(N * K // 2, DPAD)
    a0 = _mlp(0, gath0, idx0, s, g, *wb)
    a1 = _mlp(1, gath1, idx1, s, g, *wb)
    return jnp.concatenate([a0, a1], axis=0)


# speedup vs baseline: 1.4008x; 1.2371x over previous
import functools

import jax
import jax.numpy as jnp
from jax import lax
from jax.experimental import pallas as pl
from jax.experimental.pallas import tpu as pltpu
from jax.experimental.pallas import tpu_sc as plsc

N = 4096
K = 32
RB = 256
NB = N // RB
DPAD = 16


def _topk_block(s_ref, sT_ref, idx_ref, i):
    qx = s_ref[:, 0:1]
    qy = s_ref[:, 1:2]
    px = sT_ref[0:1, pl.ds(i * RB, RB)]
    py = sT_ref[1:2, pl.ds(i * RB, RB)]
    dx = qx - px
    dy = qy - py
    d = jnp.sqrt((dx * dx + 1e-6) + (dy * dy + 1e-6))
    sub = lax.broadcasted_iota(jnp.int32, (N, RB), 0)
    inf = jnp.float32(float("inf"))
    for t in range(K):
        sel = jnp.argmin(d, axis=0).astype(jnp.int32)[None, :]
        idx_ref[pl.ds(t, 1), :] = sel
        d = jnp.where(sub == sel, inf, d)


def _topk(s, sT, half):
    def body(s_ref, sT_ref, idx_ref):
        _topk_block(s_ref, sT_ref, idx_ref,
                    pl.program_id(0) + half * (NB // 2))
    out = pl.pallas_call(
        body,
        grid=(NB // 2,),
        in_specs=[
            pl.BlockSpec((N, 4), lambda i: (0, 0)),
            pl.BlockSpec((4, N), lambda i: (0, 0)),
        ],
        out_specs=pl.BlockSpec((K, RB), lambda i: (i, 0)),
        out_shape=jax.ShapeDtypeStruct((NB // 2 * K, RB), jnp.int32),
    )(s, sT)
    return out.reshape(NB // 2, K, RB).transpose(0, 2, 1).reshape(N // 2, K)


def _sc_gather(idx_flat, sT_flat, half):
    info = plsc.get_sparse_core_info()
    nc, ns = info.num_cores, info.num_subcores
    nw = nc * ns
    b = idx_flat.shape[0]
    bpw = b // nw
    nchunk = bpw // 16
    mesh = plsc.VectorSubcoreMesh(core_axis_name="c", subcore_axis_name="s")

    @functools.partial(
        pl.kernel, mesh=mesh,
        out_type=jax.ShapeDtypeStruct((b * DPAD,), jnp.float32),
        scratch_types=[
            pltpu.VMEM((4 * N,), jnp.float32),
            pltpu.VMEM((bpw,), jnp.int32),
            pltpu.VMEM((bpw * DPAD,), jnp.float32),
        ],
        compiler_params=pltpu.CompilerParams(needs_layout_passes=False),
    )
    def k(sT_hbm, idx_hbm, out_hbm, tab_v, idx_v, out_v):
        wid = lax.axis_index("s") * nc + lax.axis_index("c")
        base = wid * bpw
        pltpu.sync_copy(sT_hbm, tab_v)
        pltpu.sync_copy(idx_hbm.at[pl.ds(base, bpw)], idx_v)
        lanes = lax.broadcasted_iota(jnp.int32, (16,), 0)
        zeros = jnp.zeros((16,), jnp.float32)
        ones = zeros + 1.0

        def chunk(q, carry):
            cb = q * 16
            row = (half * b + base + cb) >> 5
            rowv = lanes * 0 + row
            idxv = idx_v[pl.ds(cb, 16)]
            pvec = (cb + lanes) * DPAD
            rel = []
            for c in range(4):
                sj = plsc.load_gather(tab_v, [idxv + c * N])
                si = plsc.load_gather(tab_v, [rowv + c * N])
                r = si - sj
                rel.append(r)
                plsc.store_scatter(out_v, [pvec + c], r)
            eye = jnp.where(idxv == rowv, ones, zeros)
            plsc.store_scatter(out_v, [pvec + 4], eye)
            msq = rel[0] * rel[0] + rel[1] * rel[1]
            mrad = jnp.where(msq < 1.0, ones, zeros)
            plsc.store_scatter(out_v, [pvec + 5], mrad)
            return carry

        lax.fori_loop(0, nchunk, chunk, 0)
        pltpu.sync_copy(out_v, out_hbm.at[pl.ds(base * DPAD, bpw * DPAD)])

    return k(sT_flat, idx_flat)


def _mlp_body(half, gath_ref, idx_ref, s_ref, g_ref,
              w1t_ref, b1_ref, w2t_ref, b2_ref,
              wf1t_ref, bf1_ref, wf2t_ref, bf2_ref,
              wf3t_ref, bf3_ref, wf4t_ref, bf4_ref, out_ref):
    del half, idx_ref
    s_blk = s_ref[...]
    g_blk = g_ref[...]
    gath = gath_ref[...].reshape(RB, K, DPAD)
    mask = gath[:, :, 5]
    x2 = gath[:, :, 0:5].reshape(RB * K, 5)
    h1 = jax.nn.relu(jnp.dot(x2, w1t_ref[...],
                             preferred_element_type=jnp.float32) + b1_ref[...])
    h2 = jax.nn.relu(jnp.dot(h1, w2t_ref[...],
                             preferred_element_type=jnp.float32) + b2_ref[...])
    hm = jnp.max(h2.reshape(RB, K, 128) * mask[:, :, None], axis=1)

    sg = s_blk[:, 0:2] - g_blk
    sv = s_blk[:, 2:4]
    feat = jnp.concatenate([hm, sg, sv], axis=1)
    f = jax.nn.relu(jnp.dot(feat, wf1t_ref[...],
                            preferred_element_type=jnp.float32) + bf1_ref[...])
    f = jax.nn.relu(jnp.dot(f, wf2t_ref[...],
                            preferred_element_type=jnp.float32) + bf2_ref[...])
    f = jax.nn.relu(jnp.dot(f, wf3t_ref[...],
                            preferred_element_type=jnp.float32) + bf3_ref[...])
    f = jnp.dot(f, wf4t_ref[...],
                preferred_element_type=jnp.float32) + bf4_ref[...]
    kk = 2.0 * jax.nn.sigmoid(f) + 0.2
    ax = -(kk[:, 0:1] * sg[:, 0:1] + kk[:, 1:2] * sv[:, 0:1])
    ay = -(kk[:, 2:3] * sg[:, 1:2] + kk[:, 3:4] * sv[:, 1:2])
    out_ref[...] = jnp.concatenate([ax, ay], axis=1)


def _mlp(half, gath, idx, s, g, *wb):
    full = lambda shape: pl.BlockSpec(shape, lambda i: tuple(0 for _ in shape))
    wb_specs = [full(w.shape) for w in wb]
    hb = NB // 2
    return pl.pallas_call(
        functools.partial(_mlp_body, half),
        grid=(hb,),
        in_specs=[
            pl.BlockSpec((RB * K, DPAD), lambda i: (i, 0)),
            pl.BlockSpec((RB, K), lambda i: (i, 0)),
            pl.BlockSpec((RB, 4), lambda i, h=half: (i + h * hb, 0)),
            pl.BlockSpec((RB, 2), lambda i, h=half: (i + h * hb, 0)),
        ] + wb_specs,
        out_specs=pl.BlockSpec((RB, 2), lambda i: (i, 0)),
        out_shape=jax.ShapeDtypeStruct((N // 2, 2), jnp.float32),
    )(gath, idx, s, g, *wb)


def kernel(s, g, W1, b1, W2, b2, Wf1, bf1, Wf2, bf2, Wf3, bf3, Wf4, bf4):
    sT = s.T
    sT_flat = sT.reshape(-1)
    wb = (W1.T, b1[None, :], W2.T, b2[None, :],
          Wf1.T, bf1[None, :], Wf2.T, bf2[None, :],
          Wf3.T, bf3[None, :], Wf4.T, bf4[None, :])
    idx0 = _topk(s, sT, 0)
    gath0 = _sc_gather(idx0.reshape(-1), sT_flat, 0).reshape(N * K // 2, DPAD)
    idx1 = _topk(s, sT, 1)
    gath1 = _sc_gather(idx1.reshape(-1), sT_flat, 1).reshape(N * K // 2, DPAD)
    a0 = _mlp(0, gath0, idx0, s, g, *wb)
    a1 = _mlp(1, gath1, idx1, s, g, *wb)
    return jnp.concatenate([a0, a1], axis=0)
